# baseline (speedup 1.0000x reference)
import functools

import numpy as np
import jax
import jax.numpy as jnp
from jax import lax
from jax.experimental import pallas as pl
from jax.experimental.pallas import tpu as pltpu
from jax.experimental.pallas import tpu_sc as plsc

_B, _K, _V = 1024, 20, 1000
_NC, _NS, _L = 2, 16, 16
_NW = _NC * _NS
_UPW = _B // _NW
_NPAIR = _K * (_K - 1) // 2
_PPAD = 192
_CPU = _PPAD // _L
_RPU = 2
_RLEN = _PPAD // _RPU
_CPR = _RLEN // _L
_ROWS = _UPW * _RPU
_HROWS = _ROWS // 2
_SCALE = np.float32(1.0 / (_B * _K * (_K - 1)))


_THRESH = [a * _K - a * (a + 1) // 2 for a in range(1, _K)]


def _make_sc_kernel():
    mesh = plsc.VectorSubcoreMesh(core_axis_name="c", subcore_axis_name="s")

    @functools.partial(
        pl.kernel,
        mesh=mesh,
        compiler_params=pltpu.CompilerParams(needs_layout_passes=False),
        out_type=jax.ShapeDtypeStruct((_NW * _L,), jnp.float32),
        scratch_types=[
            pltpu.VMEM((_UPW * _K,), jnp.int32),
            pltpu.VMEM((_PPAD,), jnp.int32),
            pltpu.VMEM((_PPAD,), jnp.int32),
            pltpu.VMEM((_ROWS, _RLEN), jnp.int32),
            pltpu.VMEM((_ROWS, _RLEN), jnp.float32),
            pltpu.VMEM((_L,), jnp.float32),
            pltpu.SemaphoreType.DMA,
            pltpu.SemaphoreType.DMA,
            pltpu.SemaphoreType.DMA,
            pltpu.SemaphoreType.DMA,
        ],
    )
    def ild_kernel(rec_hbm, d_hbm, out_hbm,
                   rec_v, pa_v, pc_v, idx_v, val_v, acc_v,
                   sem_a, sem_b, sem_c, sem_d):
        wid = lax.axis_index("s") * _NC + lax.axis_index("c")
        pltpu.sync_copy(rec_hbm.at[pl.ds(wid * (_UPW * _K), _UPW * _K)],
                        rec_v)

        lane = lax.iota(jnp.int32, _L)
        one = jnp.ones((_L,), jnp.int32)
        zero = jnp.zeros((_L,), jnp.int32)
        for t in range(_CPU):
            p = lane + t * _L
            a = zero
            for th in _THRESH:
                a = a + jnp.where(p >= th, one, zero)
            c = p - (a * _K - (a * (a + 1)) // 2) + a + 1
            c = jnp.minimum(c, _K - 1)
            pa_v[pl.ds(t * _L, _L)] = a
            pc_v[pl.ds(t * _L, _L)] = c

        def build_user(u, sem):
            base = u * _K
            for r in range(_RPU):
                for tc in range(_CPR):
                    t = r * _CPR + tc
                    ia = plsc.load_gather(rec_v, [pa_v[pl.ds(t * _L, _L)]
                                                  + base])
                    ic = plsc.load_gather(rec_v, [pc_v[pl.ds(t * _L, _L)]
                                                  + base])
                    idx_v[u * _RPU + r, pl.ds(tc * _L, _L)] = ia * _V + ic
                pltpu.make_async_copy(
                    d_hbm.at[idx_v.at[u * _RPU + r]],
                    val_v.at[u * _RPU + r], sem).start()

        sems = (sem_a, sem_b, sem_c, sem_d)
        ug = _UPW // 4
        for g in range(4):
            lax.fori_loop(g * ug, (g + 1) * ug,
                          lambda u, c, s=sems[g]: (build_user(u, s), c)[1], 0)

        def drain(sem, lo, hi):
            for j in range(lo, hi):
                pltpu.make_async_copy(d_hbm.at[pl.ds(0, _RLEN)],
                                      val_v.at[j], sem).wait()

        tailmask = jnp.where(lane + (_CPU - 1) * _L < _NPAIR,
                             jnp.float32(1.0), jnp.float32(0.0))

        def red(j, acc):
            for tc in range(_CPR):
                v = val_v[j, pl.ds(tc * _L, _L)]
                if tc == _CPR - 1:
                    v = jnp.where((j % 2) == 1, v * tailmask, v)
                acc = acc + v
            return acc

        rg = _ROWS // 4
        acc = jnp.zeros((_L,), jnp.float32)
        for g in range(4):
            drain(sems[g], g * rg, (g + 1) * rg)
            acc = lax.fori_loop(g * rg, (g + 1) * rg, red, acc)
        acc_v[...] = acc * _SCALE
        pltpu.sync_copy(acc_v, out_hbm.at[pl.ds(wid * _L, _L)])

    return ild_kernel


_SC_KERNEL = _make_sc_kernel()


def kernel(recommendations, distance_matrix):
    rec = recommendations.astype(jnp.int32).reshape(-1)
    dflat = distance_matrix.reshape(-1)
    partials = _SC_KERNEL(rec, dflat)
    return jnp.sum(partials)


# speedup vs baseline: 1.0015x; 1.0015x over previous
import functools

import numpy as np
import jax
import jax.numpy as jnp
from jax import lax
from jax.experimental import pallas as pl
from jax.experimental.pallas import tpu as pltpu
from jax.experimental.pallas import tpu_sc as plsc

_B, _K, _V = 1024, 20, 1000
_NC, _NS, _L = 2, 16, 16
_NW = _NC * _NS
_UPW = _B // _NW
_NPAIR = _K * (_K - 1) // 2
_PPAD = 192
_CPU = _PPAD // _L
_RPU = 2
_RLEN = _PPAD // _RPU
_CPR = _RLEN // _L
_ROWS = _UPW * _RPU
_HROWS = _ROWS // 2
_SCALE = np.float32(1.0 / (_B * _K * (_K - 1)))


_THRESH = [a * _K - a * (a + 1) // 2 for a in range(1, _K)]


def _make_sc_kernel():
    mesh = plsc.VectorSubcoreMesh(core_axis_name="c", subcore_axis_name="s")

    @functools.partial(
        pl.kernel,
        mesh=mesh,
        compiler_params=pltpu.CompilerParams(needs_layout_passes=False),
        out_type=jax.ShapeDtypeStruct((_NW * _L,), jnp.float32),
        scratch_types=[
            pltpu.VMEM((_UPW * _K,), jnp.int32),
            pltpu.VMEM((_PPAD,), jnp.int32),
            pltpu.VMEM((_PPAD,), jnp.int32),
            pltpu.VMEM((_ROWS, _RLEN), jnp.int32),
            pltpu.VMEM((_ROWS, _RLEN), jnp.float32),
            pltpu.VMEM((_L,), jnp.float32),
            pltpu.SemaphoreType.DMA,
            pltpu.SemaphoreType.DMA,
        ],
    )
    def ild_kernel(rec_hbm, d_hbm, out_hbm,
                   rec_v, pa_v, pc_v, idx_v, val_v, acc_v, sem_a, sem_b):
        wid = lax.axis_index("s") * _NC + lax.axis_index("c")
        pltpu.sync_copy(rec_hbm.at[pl.ds(wid * (_UPW * _K), _UPW * _K)],
                        rec_v)

        lane = lax.iota(jnp.int32, _L)
        one = jnp.ones((_L,), jnp.int32)
        zero = jnp.zeros((_L,), jnp.int32)
        for t in range(_CPU):
            p = lane + t * _L
            a = zero
            for th in _THRESH:
                a = a + jnp.where(p >= th, one, zero)
            c = p - (a * _K - (a * (a + 1)) // 2) + a + 1
            c = jnp.minimum(c, _K - 1)
            pa_v[pl.ds(t * _L, _L)] = a
            pc_v[pl.ds(t * _L, _L)] = c

        def build_user(u, sem):
            base = u * _K
            for r in range(_RPU):
                for tc in range(_CPR):
                    t = r * _CPR + tc
                    ia = plsc.load_gather(rec_v, [pa_v[pl.ds(t * _L, _L)]
                                                  + base])
                    ic = plsc.load_gather(rec_v, [pc_v[pl.ds(t * _L, _L)]
                                                  + base])
                    idx_v[u * _RPU + r, pl.ds(tc * _L, _L)] = ia * _V + ic
                pltpu.make_async_copy(
                    d_hbm.at[idx_v.at[u * _RPU + r]],
                    val_v.at[u * _RPU + r], sem).start()

        lax.fori_loop(0, _UPW // 2,
                      lambda u, c: (build_user(u, sem_a), c)[1], 0)
        lax.fori_loop(_UPW // 2, _UPW,
                      lambda u, c: (build_user(u, sem_b), c)[1], 0)

        def drain(sem, lo, hi):
            for j in range(lo, hi):
                pltpu.make_async_copy(d_hbm.at[pl.ds(0, _RLEN)],
                                      val_v.at[j], sem).wait()

        tailmask = jnp.where(lane + (_CPU - 1) * _L < _NPAIR,
                             jnp.float32(1.0), jnp.float32(0.0))

        def red(j, acc):
            for tc in range(_CPR):
                v = val_v[j, pl.ds(tc * _L, _L)]
                if tc == _CPR - 1:
                    v = jnp.where((j % 2) == 1, v * tailmask, v)
                acc = acc + v
            return acc

        drain(sem_a, 0, _HROWS)
        acc = lax.fori_loop(0, _HROWS, red, jnp.zeros((_L,), jnp.float32))
        drain(sem_b, _HROWS, _ROWS)
        acc = lax.fori_loop(_HROWS, _ROWS, red, acc)
        acc_v[...] = acc * _SCALE
        pltpu.sync_copy(acc_v, out_hbm.at[pl.ds(wid * _L, _L)])

    return ild_kernel


_SC_KERNEL = _make_sc_kernel()


def kernel(recommendations, distance_matrix):
    rec = recommendations.astype(jnp.int32).reshape(-1)
    dflat = distance_matrix.reshape(-1)
    partials = _SC_KERNEL(rec, dflat)
    return jnp.sum(partials)


# speedup vs baseline: 1.0021x; 1.0006x over previous
import functools

import numpy as np
import jax
import jax.numpy as jnp
from jax import lax
from jax.experimental import pallas as pl
from jax.experimental.pallas import tpu as pltpu
from jax.experimental.pallas import tpu_sc as plsc

_B, _K, _V = 1024, 20, 1000
_NC, _NS, _L = 2, 16, 16
_NW = _NC * _NS
_UPW = _B // _NW
_NPAIR = _K * (_K - 1) // 2
_PPAD = 192
_CPU = _PPAD // _L
_RPU = 3
_RLEN = _PPAD // _RPU
_CPR = _RLEN // _L
_ROWS = _UPW * _RPU
_HROWS = _ROWS // 2
_SCALE = np.float32(1.0 / (_B * _K * (_K - 1)))


_THRESH = [a * _K - a * (a + 1) // 2 for a in range(1, _K)]


def _make_sc_kernel():
    mesh = plsc.VectorSubcoreMesh(core_axis_name="c", subcore_axis_name="s")

    @functools.partial(
        pl.kernel,
        mesh=mesh,
        compiler_params=pltpu.CompilerParams(needs_layout_passes=False),
        out_type=jax.ShapeDtypeStruct((_NW * _L,), jnp.float32),
        scratch_types=[
            pltpu.VMEM((_UPW * _K,), jnp.int32),
            pltpu.VMEM((_PPAD,), jnp.int32),
            pltpu.VMEM((_PPAD,), jnp.int32),
            pltpu.VMEM((_ROWS, _RLEN), jnp.int32),
            pltpu.VMEM((_ROWS, _RLEN), jnp.float32),
            pltpu.VMEM((_L,), jnp.float32),
            pltpu.SemaphoreType.DMA,
            pltpu.SemaphoreType.DMA,
        ],
    )
    def ild_kernel(rec_hbm, d_hbm, out_hbm,
                   rec_v, pa_v, pc_v, idx_v, val_v, acc_v, sem_a, sem_b):
        wid = lax.axis_index("s") * _NC + lax.axis_index("c")
        pltpu.sync_copy(rec_hbm.at[pl.ds(wid * (_UPW * _K), _UPW * _K)],
                        rec_v)

        lane = lax.iota(jnp.int32, _L)
        one = jnp.ones((_L,), jnp.int32)
        zero = jnp.zeros((_L,), jnp.int32)
        for t in range(_CPU):
            p = lane + t * _L
            a = zero
            for th in _THRESH:
                a = a + jnp.where(p >= th, one, zero)
            c = p - (a * _K - (a * (a + 1)) // 2) + a + 1
            c = jnp.minimum(c, _K - 1)
            pa_v[pl.ds(t * _L, _L)] = a
            pc_v[pl.ds(t * _L, _L)] = c

        def build_user(u, sem):
            base = u * _K
            for r in range(_RPU):
                for tc in range(_CPR):
                    t = r * _CPR + tc
                    ia = plsc.load_gather(rec_v, [pa_v[pl.ds(t * _L, _L)]
                                                  + base])
                    ic = plsc.load_gather(rec_v, [pc_v[pl.ds(t * _L, _L)]
                                                  + base])
                    idx_v[u * _RPU + r, pl.ds(tc * _L, _L)] = ia * _V + ic
                pltpu.make_async_copy(
                    d_hbm.at[idx_v.at[u * _RPU + r]],
                    val_v.at[u * _RPU + r], sem).start()

        lax.fori_loop(0, _UPW // 2,
                      lambda u, c: (build_user(u, sem_a), c)[1], 0)
        lax.fori_loop(_UPW // 2, _UPW,
                      lambda u, c: (build_user(u, sem_b), c)[1], 0)

        def drain(sem, lo, hi):
            for j in range(lo, hi):
                pltpu.make_async_copy(d_hbm.at[pl.ds(0, _RLEN)],
                                      val_v.at[j], sem).wait()

        tailmask = jnp.where(lane + (_CPU - 1) * _L < _NPAIR,
                             jnp.float32(1.0), jnp.float32(0.0))

        def red(j, acc):
            for tc in range(_CPR):
                v = val_v[j, pl.ds(tc * _L, _L)]
                if tc == _CPR - 1:
                    v = jnp.where((j % _RPU) == _RPU - 1, v * tailmask, v)
                acc = acc + v
            return acc

        drain(sem_a, 0, _HROWS)
        acc = lax.fori_loop(0, _HROWS, red, jnp.zeros((_L,), jnp.float32))
        drain(sem_b, _HROWS, _ROWS)
        acc = lax.fori_loop(_HROWS, _ROWS, red, acc)
        acc_v[...] = acc * _SCALE
        pltpu.sync_copy(acc_v, out_hbm.at[pl.ds(wid * _L, _L)])

    return ild_kernel


_SC_KERNEL = _make_sc_kernel()


def kernel(recommendations, distance_matrix):
    rec = recommendations.astype(jnp.int32).reshape(-1)
    dflat = distance_matrix.reshape(-1)
    partials = _SC_KERNEL(rec, dflat)
    return jnp.sum(partials)
